# baseline (speedup 1.0000x reference)
import jax
import jax.numpy as jnp
from jax.experimental import pallas as pl
from jax.experimental.pallas import tpu as pltpu

_LRELU_SLOPE = 0.1
_BN_EPS = 1e-5
_VMEM_LIMIT = 56 * 1024 * 1024


def _conv_rows(planes, w_ref, b_ref, h, w, c_out):
    m = h * w
    cols = []
    for j in range(3):
        for p in planes:
            cols.append(p[:, j:j + w, :].reshape(m, -1))
    patches = jnp.concatenate(cols, axis=-1)
    y3 = jnp.dot(patches, w_ref[...], preferred_element_type=jnp.float32)
    return _row_epilogue(y3, b_ref, m, w, c_out)


def _row_epilogue(y3, b_ref, m, w, c_out):
    top = jnp.pad(y3[:m - w, 0 * c_out:1 * c_out], ((w, 0), (0, 0)))
    mid = y3[:, 1 * c_out:2 * c_out]
    bot = jnp.pad(y3[w:, 2 * c_out:3 * c_out], ((0, w), (0, 0)))
    return top + mid + bot + b_ref[...]


def _up_conv1_stats_kernel(fu_ref, fd_ref, wu_ref, bu_ref, wcu_ref, wcd_ref,
                           b_ref, y_ref, s_ref, q_ref, up_scr):
    _, c, m = fd_ref.shape
    _, h2, w2, _ = y_ref.shape
    h, w = h2, w2
    sh, sw = h // 2, w // 2
    x = fu_ref[0].astype(jnp.bfloat16)
    acc_up = jax.lax.dot_general(
        x, wu_ref[...], (((0,), (0,)), ((), ())),
        preferred_element_type=jnp.float32)
    acc_up = acc_up + bu_ref[...]
    up_scr[:, 0:1, :] = jnp.zeros((h, 1, c), jnp.float32)
    up_scr[:, w + 1:w + 2, :] = jnp.zeros((h, 1, c), jnp.float32)
    for dy in range(2):
        for dx in range(2):
            t = 2 * dy + dx
            tap = acc_up[:, t * c:(t + 1) * c]
            up_scr[dy::2, 1 + dx:1 + w:2, :] = tap.reshape(sh, sw, c)
    up_p = up_scr[...].astype(jnp.bfloat16)
    cols = [up_p[:, j:j + w, :].reshape(m, c) for j in range(3)]
    y3 = jnp.dot(jnp.concatenate(cols, axis=-1), wcu_ref[...],
                 preferred_element_type=jnp.float32)
    fdc = fd_ref[0].astype(jnp.bfloat16)
    wpos = jax.lax.broadcasted_iota(jnp.int32, (1, m), 1) % w
    left = jnp.pad(fdc[:, :m - 1], ((0, 0), (1, 0)))
    left = jnp.where(wpos != 0, left, jnp.bfloat16(0))
    right = jnp.pad(fdc[:, 1:], ((0, 0), (0, 1)))
    right = jnp.where(wpos != w - 1, right, jnp.bfloat16(0))
    fd_t = jnp.concatenate([left, fdc, right], axis=0)
    y3 = y3 + jax.lax.dot_general(
        fd_t, wcd_ref[...], (((0,), (0,)), ((), ())),
        preferred_element_type=jnp.float32)
    acc = _row_epilogue(y3, b_ref, m, w, c)
    y_ref[0] = acc.reshape(h, w, -1).astype(y_ref.dtype)
    s_ref[0] = jnp.sum(acc, axis=0, keepdims=True)
    q_ref[0] = jnp.sum(acc * acc, axis=0, keepdims=True)


def _bn_conv2_stats_kernel(y1_ref, sc_ref, sh_ref, w_ref, b_ref,
                           y_ref, s_ref, q_ref):
    _, h, w, c = y1_ref.shape
    z = (y1_ref[0].astype(jnp.float32) * sc_ref[...].reshape(1, 1, c)
         + sh_ref[...].reshape(1, 1, c))
    a = jnp.where(z >= 0, z, _LRELU_SLOPE * z).astype(jnp.bfloat16)
    a_p = jnp.pad(a, ((0, 0), (1, 1), (0, 0)))
    acc = _conv_rows([a_p], w_ref, b_ref, h, w, c)
    y_ref[0] = acc.reshape(h, w, -1).astype(y_ref.dtype)
    s_ref[0] = jnp.sum(acc, axis=0, keepdims=True)
    q_ref[0] = jnp.sum(acc * acc, axis=0, keepdims=True)


def _bn_lrelu_out_kernel(y_ref, sc_ref, sh_ref, o_ref):
    c = y_ref.shape[-1]
    z = (y_ref[0].astype(jnp.float32) * sc_ref[...].reshape(1, 1, c)
         + sh_ref[...].reshape(1, 1, c))
    o_ref[0] = jnp.where(z >= 0, z, _LRELU_SLOPE * z)


def _scale_shift(s_part, q_part, gamma, beta, count):
    ssum = jnp.sum(s_part[:, 0, :], axis=0)
    qsum = jnp.sum(q_part[:, 0, :], axis=0)
    mean = ssum / count
    var = jnp.maximum(qsum / count - mean * mean, 0.0)
    scale = gamma / jnp.sqrt(var + _BN_EPS)
    shift = beta - mean * scale
    c = gamma.shape[0]
    return scale.reshape(1, c).astype(jnp.float32), \
        shift.reshape(1, c).astype(jnp.float32)


def _conv_weight(w_hwio):
    kh, kw, cin, cout = w_hwio.shape
    return jnp.transpose(w_hwio, (1, 2, 0, 3)).reshape(kw * cin, kh * cout)


def _params(sem):
    return pltpu.CompilerParams(dimension_semantics=(sem,),
                                vmem_limit_bytes=_VMEM_LIMIT)


def kernel(from_down, from_up, up_w, up_b, w1, b1, gamma1, beta1,
           w2, b2, gamma2, beta2):
    n, cin, h, w = from_up.shape
    cout = up_w.shape[-1]
    hh, ww = 2 * h, 2 * w
    bf = jnp.bfloat16

    fu = from_up.reshape(n, cin, h * w)
    fdm = from_down.reshape(n, cout, hh * ww)
    wup = jnp.transpose(up_w, (2, 0, 1, 3)).reshape(cin, 4 * cout)
    bup = jnp.tile(up_b, 4).reshape(1, 4 * cout).astype(jnp.float32)

    w1u = _conv_weight(w1[:, :, :cout, :]).astype(bf)
    w1d = _conv_weight(w1[:, :, cout:, :]).astype(bf)
    b1r = b1.reshape(1, cout).astype(jnp.float32)
    y1, s1, q1 = pl.pallas_call(
        _up_conv1_stats_kernel,
        out_shape=(
            jax.ShapeDtypeStruct((n, hh, ww, cout), bf),
            jax.ShapeDtypeStruct((n, 1, cout), jnp.float32),
            jax.ShapeDtypeStruct((n, 1, cout), jnp.float32),
        ),
        grid=(n,),
        in_specs=[
            pl.BlockSpec((1, cin, h * w), lambda i: (i, 0, 0)),
            pl.BlockSpec((1, cout, hh * ww), lambda i: (i, 0, 0)),
            pl.BlockSpec((cin, 4 * cout), lambda i: (0, 0)),
            pl.BlockSpec((1, 4 * cout), lambda i: (0, 0)),
            pl.BlockSpec((3 * cout, 3 * cout), lambda i: (0, 0)),
            pl.BlockSpec((3 * cout, 3 * cout), lambda i: (0, 0)),
            pl.BlockSpec((1, cout), lambda i: (0, 0)),
        ],
        out_specs=[
            pl.BlockSpec((1, hh, ww, cout), lambda i: (i, 0, 0, 0)),
            pl.BlockSpec((1, 1, cout), lambda i: (i, 0, 0)),
            pl.BlockSpec((1, 1, cout), lambda i: (i, 0, 0)),
        ],
        scratch_shapes=[pltpu.VMEM((hh, ww + 2, cout), jnp.float32)],
        compiler_params=_params("parallel"),
    )(fu, fdm, wup.astype(bf), bup, w1u, w1d, b1r)

    count = jnp.float32(n * hh * ww)
    sc1, sh1 = _scale_shift(s1, q1, gamma1, beta1, count)

    w2r = _conv_weight(w2).astype(bf)
    b2r = b2.reshape(1, cout).astype(jnp.float32)
    y2, s2, q2 = pl.pallas_call(
        _bn_conv2_stats_kernel,
        out_shape=(
            jax.ShapeDtypeStruct((n, hh, ww, cout), bf),
            jax.ShapeDtypeStruct((n, 1, cout), jnp.float32),
            jax.ShapeDtypeStruct((n, 1, cout), jnp.float32),
        ),
        grid=(n,),
        in_specs=[
            pl.BlockSpec((1, hh, ww, cout), lambda i: (i, 0, 0, 0)),
            pl.BlockSpec((1, cout), lambda i: (0, 0)),
            pl.BlockSpec((1, cout), lambda i: (0, 0)),
            pl.BlockSpec((3 * cout, 3 * cout), lambda i: (0, 0)),
            pl.BlockSpec((1, cout), lambda i: (0, 0)),
        ],
        out_specs=[
            pl.BlockSpec((1, hh, ww, cout), lambda i: (i, 0, 0, 0)),
            pl.BlockSpec((1, 1, cout), lambda i: (i, 0, 0)),
            pl.BlockSpec((1, 1, cout), lambda i: (i, 0, 0)),
        ],
        compiler_params=_params("parallel"),
    )(y1, sc1, sh1, w2r, b2r)

    sc2, sh2 = _scale_shift(s2, q2, gamma2, beta2, count)

    out = pl.pallas_call(
        _bn_lrelu_out_kernel,
        out_shape=jax.ShapeDtypeStruct((n, hh, ww, cout), jnp.float32),
        grid=(n,),
        in_specs=[
            pl.BlockSpec((1, hh, ww, cout), lambda i: (i, 0, 0, 0)),
            pl.BlockSpec((1, cout), lambda i: (0, 0)),
            pl.BlockSpec((1, cout), lambda i: (0, 0)),
        ],
        out_specs=pl.BlockSpec((1, hh, ww, cout), lambda i: (i, 0, 0, 0)),
        compiler_params=_params("parallel"),
    )(y2, sc2, sh2)

    return jnp.transpose(out, (0, 3, 1, 2))


# speedup vs baseline: 1.2263x; 1.2263x over previous
import jax
import jax.numpy as jnp
from jax.experimental import pallas as pl
from jax.experimental.pallas import tpu as pltpu

_LRELU_SLOPE = 0.1
_BN_EPS = 1e-5
_VMEM_LIMIT = 56 * 1024 * 1024


def _conv_rows(planes, w_ref, b_ref, h, w, c_out):
    m = h * w
    cols = []
    for j in range(3):
        for p in planes:
            cols.append(p[:, j:j + w, :].reshape(m, -1))
    patches = jnp.concatenate(cols, axis=-1)
    y3 = jnp.dot(patches, w_ref[...], preferred_element_type=jnp.float32)
    return _row_epilogue(y3, b_ref, m, w, c_out)


def _row_epilogue(y3, b_ref, m, w, c_out):
    top = jnp.pad(y3[:m - w, 0 * c_out:1 * c_out], ((w, 0), (0, 0)))
    mid = y3[:, 1 * c_out:2 * c_out]
    bot = jnp.pad(y3[w:, 2 * c_out:3 * c_out], ((0, w), (0, 0)))
    return top + mid + bot + b_ref[...]


def _up_conv1_stats_kernel(fu_ref, fd_ref, wu_ref, bu_ref, w_ref, b_ref,
                           y_ref, s_ref, q_ref, up_scr):
    _, h, w, c = fd_ref.shape
    sh, sw = h // 2, w // 2
    x = fu_ref[0].astype(jnp.bfloat16)
    acc_up = jax.lax.dot_general(
        x, wu_ref[...], (((0,), (0,)), ((), ())),
        preferred_element_type=jnp.float32)
    acc_up = acc_up + bu_ref[...]
    up_scr[:, 0:1, :] = jnp.zeros((h, 1, c), jnp.float32)
    up_scr[:, w + 1:w + 2, :] = jnp.zeros((h, 1, c), jnp.float32)
    for dy in range(2):
        for dx in range(2):
            t = 2 * dy + dx
            tap = acc_up[:, t * c:(t + 1) * c]
            up_scr[dy::2, 1 + dx:1 + w:2, :] = tap.reshape(sh, sw, c)
    up_p = up_scr[...].astype(jnp.bfloat16)
    fd_p = jnp.pad(fd_ref[0], ((0, 0), (1, 1), (0, 0)))
    acc = _conv_rows([up_p, fd_p], w_ref, b_ref, h, w, c)
    y_ref[0] = acc.reshape(h, w, -1).astype(y_ref.dtype)
    s_ref[0] = jnp.sum(acc, axis=0, keepdims=True)
    q_ref[0] = jnp.sum(acc * acc, axis=0, keepdims=True)


def _bn_conv2_stats_kernel(y1_ref, sc_ref, sh_ref, w_ref, b_ref,
                           y_ref, s_ref, q_ref):
    _, h, w, c = y1_ref.shape
    z = (y1_ref[0].astype(jnp.float32) * sc_ref[...].reshape(1, 1, c)
         + sh_ref[...].reshape(1, 1, c))
    a = jnp.where(z >= 0, z, _LRELU_SLOPE * z).astype(jnp.bfloat16)
    a_p = jnp.pad(a, ((0, 0), (1, 1), (0, 0)))
    acc = _conv_rows([a_p], w_ref, b_ref, h, w, c)
    y_ref[0] = acc.reshape(h, w, -1).astype(y_ref.dtype)
    s_ref[0] = jnp.sum(acc, axis=0, keepdims=True)
    q_ref[0] = jnp.sum(acc * acc, axis=0, keepdims=True)


def _bn_lrelu_out_kernel(y_ref, sc_ref, sh_ref, o_ref):
    c = y_ref.shape[-1]
    z = (y_ref[0].astype(jnp.float32) * sc_ref[...].reshape(1, 1, c)
         + sh_ref[...].reshape(1, 1, c))
    o_ref[0] = jnp.where(z >= 0, z, _LRELU_SLOPE * z)


def _scale_shift(s_part, q_part, gamma, beta, count):
    ssum = jnp.sum(s_part[:, 0, :], axis=0)
    qsum = jnp.sum(q_part[:, 0, :], axis=0)
    mean = ssum / count
    var = jnp.maximum(qsum / count - mean * mean, 0.0)
    scale = gamma / jnp.sqrt(var + _BN_EPS)
    shift = beta - mean * scale
    c = gamma.shape[0]
    return scale.reshape(1, c).astype(jnp.float32), \
        shift.reshape(1, c).astype(jnp.float32)


def _conv_weight(w_hwio):
    kh, kw, cin, cout = w_hwio.shape
    return jnp.transpose(w_hwio, (1, 2, 0, 3)).reshape(kw * cin, kh * cout)


def _params(sem):
    return pltpu.CompilerParams(dimension_semantics=(sem,),
                                vmem_limit_bytes=_VMEM_LIMIT)


def kernel(from_down, from_up, up_w, up_b, w1, b1, gamma1, beta1,
           w2, b2, gamma2, beta2):
    n, cin, h, w = from_up.shape
    cout = up_w.shape[-1]
    hh, ww = 2 * h, 2 * w
    bf = jnp.bfloat16

    fu = from_up.reshape(n, cin, h * w)
    wup = jnp.transpose(up_w, (2, 0, 1, 3)).reshape(cin, 4 * cout)
    bup = jnp.tile(up_b, 4).reshape(1, 4 * cout).astype(jnp.float32)
    fd = jnp.transpose(from_down, (0, 2, 3, 1)).astype(bf)

    w1r = _conv_weight(w1).astype(bf)
    b1r = b1.reshape(1, cout).astype(jnp.float32)
    y1, s1, q1 = pl.pallas_call(
        _up_conv1_stats_kernel,
        out_shape=(
            jax.ShapeDtypeStruct((n, hh, ww, cout), bf),
            jax.ShapeDtypeStruct((n, 1, cout), jnp.float32),
            jax.ShapeDtypeStruct((n, 1, cout), jnp.float32),
        ),
        grid=(n,),
        in_specs=[
            pl.BlockSpec((1, cin, h * w), lambda i: (i, 0, 0)),
            pl.BlockSpec((1, hh, ww, cout), lambda i: (i, 0, 0, 0)),
            pl.BlockSpec((cin, 4 * cout), lambda i: (0, 0)),
            pl.BlockSpec((1, 4 * cout), lambda i: (0, 0)),
            pl.BlockSpec((3 * 2 * cout, 3 * cout), lambda i: (0, 0)),
            pl.BlockSpec((1, cout), lambda i: (0, 0)),
        ],
        out_specs=[
            pl.BlockSpec((1, hh, ww, cout), lambda i: (i, 0, 0, 0)),
            pl.BlockSpec((1, 1, cout), lambda i: (i, 0, 0)),
            pl.BlockSpec((1, 1, cout), lambda i: (i, 0, 0)),
        ],
        scratch_shapes=[pltpu.VMEM((hh, ww + 2, cout), jnp.float32)],
        compiler_params=_params("parallel"),
    )(fu, fd, wup.astype(bf), bup, w1r, b1r)

    count = jnp.float32(n * hh * ww)
    sc1, sh1 = _scale_shift(s1, q1, gamma1, beta1, count)

    w2r = _conv_weight(w2).astype(bf)
    b2r = b2.reshape(1, cout).astype(jnp.float32)
    y2, s2, q2 = pl.pallas_call(
        _bn_conv2_stats_kernel,
        out_shape=(
            jax.ShapeDtypeStruct((n, hh, ww, cout), bf),
            jax.ShapeDtypeStruct((n, 1, cout), jnp.float32),
            jax.ShapeDtypeStruct((n, 1, cout), jnp.float32),
        ),
        grid=(n,),
        in_specs=[
            pl.BlockSpec((1, hh, ww, cout), lambda i: (i, 0, 0, 0)),
            pl.BlockSpec((1, cout), lambda i: (0, 0)),
            pl.BlockSpec((1, cout), lambda i: (0, 0)),
            pl.BlockSpec((3 * cout, 3 * cout), lambda i: (0, 0)),
            pl.BlockSpec((1, cout), lambda i: (0, 0)),
        ],
        out_specs=[
            pl.BlockSpec((1, hh, ww, cout), lambda i: (i, 0, 0, 0)),
            pl.BlockSpec((1, 1, cout), lambda i: (i, 0, 0)),
            pl.BlockSpec((1, 1, cout), lambda i: (i, 0, 0)),
        ],
        compiler_params=_params("parallel"),
    )(y1, sc1, sh1, w2r, b2r)

    sc2, sh2 = _scale_shift(s2, q2, gamma2, beta2, count)

    out = pl.pallas_call(
        _bn_lrelu_out_kernel,
        out_shape=jax.ShapeDtypeStruct((n, hh, ww, cout), jnp.float32),
        grid=(n,),
        in_specs=[
            pl.BlockSpec((1, hh, ww, cout), lambda i: (i, 0, 0, 0)),
            pl.BlockSpec((1, cout), lambda i: (0, 0)),
            pl.BlockSpec((1, cout), lambda i: (0, 0)),
        ],
        out_specs=pl.BlockSpec((1, hh, ww, cout), lambda i: (i, 0, 0, 0)),
        compiler_params=_params("parallel"),
    )(y2, sc2, sh2)

    return jnp.transpose(out, (0, 3, 1, 2))
